# Initial kernel scaffold; baseline (speedup 1.0000x reference)
import jax
import jax.numpy as jnp
from jax.experimental import pallas as pl


def kernel(x, edge_index, graph_ids, Wgate, Wexp, bexp, We1, be1, We2, be2, We3, be3, We4, be4, Wp, bp):
    raise NotImplementedError("write your pallas kernel here")


# speedup vs baseline: 4.9436x; 4.9436x over previous
import functools

import jax
import jax.numpy as jnp
from jax import lax
from jax.experimental import pallas as pl
from jax.experimental.pallas import tpu as pltpu
from jax.experimental.pallas import tpu_sc as plsc

N = 10000
D = 128
E = 320000
L = 5
NEXP = 8
G = 64
H = 256
NT = 10

NW = 32
CB = 128
CH = 80
EPW = CH * CB
E_PAD = NW * EPW
NPAD = 10240
RPT = NPAD // 16
TRASH = 10100

BN = 400
NB = N // BN

@functools.cache
def _sc_kernels():
    mesh = plsc.VectorSubcoreMesh(core_axis_name="c", subcore_axis_name="s")
    deg = functools.partial(
        pl.kernel, mesh=mesh,
        out_type=jax.ShapeDtypeStruct((2 * NPAD, D), jnp.float32),
        scratch_types=[
            pltpu.VMEM((CH, CB), jnp.int32),
            pltpu.VMEM((CB, D), jnp.float32),
            pltpu.VMEM((CB, D), jnp.float32),
            pltpu.VMEM_SHARED((NPAD, D), jnp.float32),
        ],
    )(_deg_sc_body)
    agg = functools.partial(
        pl.kernel, mesh=mesh,
        out_type=jax.ShapeDtypeStruct((2 * NPAD, D), jnp.float32),
        scratch_types=[
            pltpu.VMEM((CH, CB), jnp.int32),
            pltpu.VMEM((CH, CB), jnp.int32),
            pltpu.VMEM((CB, D), jnp.float32),
            pltpu.SemaphoreType.DMA,
            pltpu.VMEM_SHARED((NPAD, D), jnp.float32),
        ],
    )(_agg_sc_body)
    return deg, agg


def _deg_sc_body(dst_hbm, ones_hbm, zeros_hbm, out_hbm, didx, ones_v, zeros_v, acc):
    cid = lax.axis_index("c")
    sid = lax.axis_index("s")
    wid = sid * 2 + cid
    pltpu.sync_copy(dst_hbm.at[wid], didx)
    pltpu.sync_copy(ones_hbm, ones_v)
    pltpu.sync_copy(zeros_hbm, zeros_v)
    for t in range(RPT // CB):
        pltpu.sync_copy(zeros_v, acc.at[pl.ds(sid * RPT + t * CB, CB)])
    plsc.subcore_barrier()

    def body(j, carry):
        pltpu.sync_copy(ones_v, acc.at[didx.at[j]], add=True)
        return carry

    lax.fori_loop(0, CH, body, 0)
    plsc.subcore_barrier()
    for t in range(RPT // CB):
        r0 = sid * RPT + t * CB
        pltpu.sync_copy(acc.at[pl.ds(r0, CB)], out_hbm.at[pl.ds(cid * NPAD + r0, CB)])


def _agg_sc_body(xs_hbm, src_hbm, dst_hbm, zeros_hbm, out_hbm, sidx, didx, rows, sem, acc):
    cid = lax.axis_index("c")
    sid = lax.axis_index("s")
    wid = sid * 2 + cid
    pltpu.sync_copy(src_hbm.at[wid], sidx)
    pltpu.sync_copy(dst_hbm.at[wid], didx)
    pltpu.sync_copy(zeros_hbm, rows)
    for t in range(RPT // CB):
        pltpu.sync_copy(rows, acc.at[pl.ds(sid * RPT + t * CB, CB)])
    plsc.subcore_barrier()

    def body(j, carry):
        pltpu.async_copy(xs_hbm.at[sidx.at[j]], rows, sem).wait()
        pltpu.sync_copy(rows, acc.at[didx.at[j]], add=True)
        return carry

    lax.fori_loop(0, CH, body, 0)
    plsc.subcore_barrier()
    for t in range(RPT // CB):
        r0 = sid * RPT + t * CB
        pltpu.sync_copy(acc.at[pl.ds(r0, CB)], out_hbm.at[pl.ds(cid * NPAD + r0, CB)])


def _xs_body(deg_ref, x_ref, xs_ref):
    dsum = deg_ref[0] + deg_ref[1]
    dinv = lax.rsqrt(jnp.maximum(dsum, 1.0))[:, 0:1]
    xs_ref[...] = x_ref[...] * dinv


_xs_call = pl.pallas_call(
    _xs_body,
    grid=(NB,),
    in_specs=[
        pl.BlockSpec((2, BN, D), lambda i: (0, i, 0)),
        pl.BlockSpec((BN, D), lambda i: (i, 0)),
    ],
    out_specs=pl.BlockSpec((BN, D), lambda i: (i, 0)),
    out_shape=jax.ShapeDtypeStruct((N, D), jnp.float32),
)


def _layer_body(h_ref, p_ref, deg_ref, wg_ref, wx_ref, bx_ref, h_out, xs_out, *, relu):
    dsum = deg_ref[0] + deg_ref[1]
    dinv = lax.rsqrt(jnp.maximum(dsum, 1.0))[:, 0:1]
    agg = (p_ref[0] + p_ref[1]) * dinv
    h = h_ref[...]
    logits = jnp.dot(h, wg_ref[...], preferred_element_type=jnp.float32)
    e_iota = lax.broadcasted_iota(jnp.int32, (BN, NEXP), 1)
    m1 = jnp.max(logits, axis=1, keepdims=True)
    i1 = jnp.min(jnp.where(logits == m1, e_iota, NEXP), axis=1, keepdims=True)
    masked = jnp.where(e_iota == i1, -jnp.inf, logits)
    m2 = jnp.max(masked, axis=1, keepdims=True)
    i2 = jnp.min(jnp.where(masked == m2, e_iota, NEXP), axis=1, keepdims=True)
    t = jnp.exp(m2 - m1)
    w1 = 1.0 / (1.0 + t)
    w2 = t / (1.0 + t)
    gates = (w1 * (e_iota == i1).astype(jnp.float32)
             + w2 * (e_iota == i2).astype(jnp.float32))
    gates_r = gates.astype(jnp.bfloat16).astype(jnp.float32)
    acc = jnp.zeros((BN, D), jnp.float32)
    for e in range(NEXP):
        eo = jnp.dot(agg, wx_ref[e], preferred_element_type=jnp.float32) + bx_ref[e]
        eo = eo.astype(jnp.bfloat16).astype(jnp.float32)
        acc = acc + gates_r[:, e:e + 1] * eo
    if relu:
        acc = jnp.maximum(acc, 0.0)
    h_out[...] = acc
    xs_out[...] = acc * dinv


def _make_layer(relu):
    return pl.pallas_call(
        functools.partial(_layer_body, relu=relu),
        grid=(NB,),
        in_specs=[
            pl.BlockSpec((BN, D), lambda i: (i, 0)),
            pl.BlockSpec((2, BN, D), lambda i: (0, i, 0)),
            pl.BlockSpec((2, BN, D), lambda i: (0, i, 0)),
            pl.BlockSpec((D, NEXP), lambda i: (0, 0)),
            pl.BlockSpec((NEXP, D, D), lambda i: (0, 0, 0)),
            pl.BlockSpec((NEXP, D), lambda i: (0, 0)),
        ],
        out_specs=[
            pl.BlockSpec((BN, D), lambda i: (i, 0)),
            pl.BlockSpec((BN, D), lambda i: (i, 0)),
        ],
        out_shape=[
            jax.ShapeDtypeStruct((N, D), jnp.float32),
            jax.ShapeDtypeStruct((N, D), jnp.float32),
        ],
    )


_layer_relu = _make_layer(True)
_layer_last = _make_layer(False)


def _pool_body(gid_ref, h_ref, w1_ref, b1_ref, w2_ref, b2_ref, w3_ref, b3_ref,
               w4_ref, b4_ref, wp_ref, bp_ref, out_logp, out_hidden, sums, cnts):
    i = pl.program_id(0)
    gcol = gid_ref[...]
    g_iota = lax.broadcasted_iota(jnp.int32, (BN, G), 1).astype(jnp.float32)
    oh = (gcol == g_iota).astype(jnp.float32)
    cdims = (((0,), (0,)), ((), ()))
    part = lax.dot_general(oh, h_ref[...], cdims,
                           preferred_element_type=jnp.float32,
                           precision=lax.Precision.HIGHEST)
    pcnt = lax.dot_general(oh, jnp.ones((BN, D), jnp.float32), cdims,
                           preferred_element_type=jnp.float32)

    @pl.when(i == 0)
    def _init():
        sums[...] = part
        cnts[...] = pcnt

    @pl.when(i > 0)
    def _accum():
        sums[...] += part
        cnts[...] += pcnt

    @pl.when(i == NB - 1)
    def _final():
        h_node = sums[...] / jnp.maximum(cnts[...], 1.0)
        hidden = jnp.maximum(
            jnp.dot(h_node, w1_ref[...], preferred_element_type=jnp.float32)
            + b1_ref[...], 0.0)
        h2 = jnp.maximum(
            jnp.dot(hidden, w2_ref[...], preferred_element_type=jnp.float32)
            + b2_ref[...], 0.0)
        h3 = jnp.maximum(
            jnp.dot(h2, w3_ref[...], preferred_element_type=jnp.float32)
            + b3_ref[...], 0.0)
        h4 = jnp.maximum(
            jnp.dot(h3, w4_ref[...], preferred_element_type=jnp.float32)
            + b4_ref[...], 0.0)
        y = jnp.dot(h4, wp_ref[...], preferred_element_type=jnp.float32) + bp_ref[...]
        m = jnp.max(y, axis=1, keepdims=True)
        lse = jnp.log(jnp.sum(jnp.exp(y - m), axis=1, keepdims=True)) + m
        out_logp[...] = y - lse
        out_hidden[...] = hidden


_pool_call = pl.pallas_call(
    _pool_body,
    grid=(NB,),
    in_specs=[
        pl.BlockSpec((BN, 1), lambda i: (i, 0)),
        pl.BlockSpec((BN, D), lambda i: (i, 0)),
        pl.BlockSpec((D, H), lambda i: (0, 0)),
        pl.BlockSpec((1, H), lambda i: (0, 0)),
        pl.BlockSpec((H, H), lambda i: (0, 0)),
        pl.BlockSpec((1, H), lambda i: (0, 0)),
        pl.BlockSpec((H, H), lambda i: (0, 0)),
        pl.BlockSpec((1, H), lambda i: (0, 0)),
        pl.BlockSpec((H, H), lambda i: (0, 0)),
        pl.BlockSpec((1, H), lambda i: (0, 0)),
        pl.BlockSpec((H, NT), lambda i: (0, 0)),
        pl.BlockSpec((1, NT), lambda i: (0, 0)),
    ],
    out_specs=[
        pl.BlockSpec((G, NT), lambda i: (0, 0)),
        pl.BlockSpec((G, H), lambda i: (0, 0)),
    ],
    out_shape=[
        jax.ShapeDtypeStruct((G, NT), jnp.float32),
        jax.ShapeDtypeStruct((G, H), jnp.float32),
    ],
    scratch_shapes=[
        pltpu.VMEM((G, D), jnp.float32),
        pltpu.VMEM((G, D), jnp.float32),
    ],
)


def kernel(x, edge_index, graph_ids, Wgate, Wexp, bexp,
           We1, be1, We2, be2, We3, be3, We4, be4, Wp, bp):
    src = edge_index[0]
    dst = edge_index[1]
    pad = E_PAD - E
    src_p = jnp.concatenate([src, jnp.zeros((pad,), jnp.int32)]).reshape(NW, CH, CB)
    dst_p = jnp.concatenate([dst, jnp.full((pad,), TRASH, jnp.int32)]).reshape(NW, CH, CB)
    ones_r = jnp.ones((CB, D), jnp.float32)
    zrows = jnp.zeros((CB, D), jnp.float32)

    deg_sc, agg_sc = _sc_kernels()
    deg_parts = deg_sc(dst_p, ones_r, zrows).reshape(2, NPAD, D)
    xs = _xs_call(deg_parts, x)

    h = x
    for l in range(L):
        parts = agg_sc(xs, src_p, dst_p, zrows).reshape(2, NPAD, D)
        layer = _layer_relu if l < L - 1 else _layer_last
        h, xs = layer(h, parts, deg_parts, Wgate[l], Wexp[l], bexp[l])

    gidf = graph_ids.astype(jnp.float32)[:, None]
    logp, hidden = _pool_call(gidf, h, We1, be1[None, :], We2, be2[None, :],
                              We3, be3[None, :], We4, be4[None, :], Wp, bp[None, :])
    return (logp, hidden)


# speedup vs baseline: 5.6514x; 1.1432x over previous
import functools

import jax
import jax.numpy as jnp
from jax import lax
from jax.experimental import pallas as pl
from jax.experimental.pallas import tpu as pltpu
from jax.experimental.pallas import tpu_sc as plsc

N = 10000
D = 128
E = 320000
L = 5
NEXP = 8
G = 64
H = 256
NT = 10

NW = 32
CB = 128
CH = 80
CHH = CH // 2
EPW = CH * CB
E_PAD = NW * EPW
NPAD = 10240
RPT = NPAD // 16
TRASH = 10100

BN = 400
NB = N // BN

@functools.cache
def _sc_kernels():
    mesh = plsc.VectorSubcoreMesh(core_axis_name="c", subcore_axis_name="s")
    deg = functools.partial(
        pl.kernel, mesh=mesh,
        out_type=jax.ShapeDtypeStruct((2 * NPAD, D), jnp.float32),
        scratch_types=[
            pltpu.VMEM((CHH, CB), jnp.int32),
            pltpu.VMEM((CB, D), jnp.float32),
            pltpu.VMEM((CB, D), jnp.float32),
            pltpu.VMEM_SHARED((NPAD, D), jnp.float32),
        ],
    )(_deg_sc_body)
    agg = functools.partial(
        pl.kernel, mesh=mesh,
        out_type=jax.ShapeDtypeStruct((2 * NPAD, D), jnp.float32),
        scratch_types=[
            pltpu.VMEM((CHH, CB), jnp.int32),
            pltpu.VMEM((CHH, CB), jnp.int32),
            pltpu.VMEM((CB, D), jnp.float32),
            pltpu.VMEM((CB, D), jnp.float32),
            pltpu.SemaphoreType.DMA,
            pltpu.SemaphoreType.DMA,
            pltpu.VMEM_SHARED((NPAD, D), jnp.float32),
        ],
    )(_agg_sc_body)
    return deg, agg


def _deg_sc_body(dst_hbm, ones_hbm, zeros_hbm, out_hbm, didx, ones_v, zeros_v, acc):
    cid = lax.axis_index("c")
    sid = lax.axis_index("s")
    wid = sid * 2 + cid
    pltpu.sync_copy(ones_hbm, ones_v)
    pltpu.sync_copy(zeros_hbm, zeros_v)
    for t in range(RPT // CB):
        pltpu.sync_copy(zeros_v, acc.at[pl.ds(sid * RPT + t * CB, CB)])
    plsc.subcore_barrier()

    def body(j, carry):
        pltpu.sync_copy(ones_v, acc.at[didx.at[j]], add=True)
        return carry

    for half in range(2):
        pltpu.sync_copy(dst_hbm.at[wid * 2 + half], didx)
        lax.fori_loop(0, CHH, body, 0)
    plsc.subcore_barrier()
    for t in range(RPT // CB):
        r0 = sid * RPT + t * CB
        pltpu.sync_copy(acc.at[pl.ds(r0, CB)], out_hbm.at[pl.ds(cid * NPAD + r0, CB)])


def _agg_sc_body(xs_hbm, src_hbm, dst_hbm, zeros_hbm, out_hbm, sidx, didx,
                 rows0, rows1, sem0, sem1, acc):
    cid = lax.axis_index("c")
    sid = lax.axis_index("s")
    wid = sid * 2 + cid
    pltpu.sync_copy(zeros_hbm, rows0)
    for t in range(RPT // CB):
        pltpu.sync_copy(rows0, acc.at[pl.ds(sid * RPT + t * CB, CB)])
    plsc.subcore_barrier()

    bufs = (rows0, rows1)
    sems = (sem0, sem1)
    for half in range(2):
        pltpu.sync_copy(src_hbm.at[wid * 2 + half], sidx)
        pltpu.sync_copy(dst_hbm.at[wid * 2 + half], didx)
        pltpu.async_copy(xs_hbm.at[sidx.at[0]], bufs[0], sems[0])

        def pair(jj, carry):
            for b in range(2):
                j = 2 * jj + b

                @pl.when(j + 1 < CHH)
                def _start_next():
                    pltpu.async_copy(xs_hbm.at[sidx.at[j + 1]], bufs[1 - b], sems[1 - b])

                pltpu.make_async_copy(xs_hbm.at[sidx.at[j]], bufs[b], sems[b]).wait()
                pltpu.sync_copy(bufs[b], acc.at[didx.at[j]], add=True)
            return carry

        lax.fori_loop(0, CHH // 2, pair, 0)
    plsc.subcore_barrier()
    for t in range(RPT // CB):
        r0 = sid * RPT + t * CB
        pltpu.sync_copy(acc.at[pl.ds(r0, CB)], out_hbm.at[pl.ds(cid * NPAD + r0, CB)])


def _xs_body(deg_ref, x_ref, xs_ref):
    dsum = deg_ref[0] + deg_ref[1]
    dinv = lax.rsqrt(jnp.maximum(dsum, 1.0))[:, 0:1]
    xs_ref[...] = x_ref[...] * dinv


_xs_call = pl.pallas_call(
    _xs_body,
    grid=(NB,),
    in_specs=[
        pl.BlockSpec((2, BN, D), lambda i: (0, i, 0)),
        pl.BlockSpec((BN, D), lambda i: (i, 0)),
    ],
    out_specs=pl.BlockSpec((BN, D), lambda i: (i, 0)),
    out_shape=jax.ShapeDtypeStruct((N, D), jnp.float32),
)


def _layer_body(h_ref, p_ref, deg_ref, wg_ref, wx_ref, bx_ref, h_out, xs_out, *, relu):
    dsum = deg_ref[0] + deg_ref[1]
    dinv = lax.rsqrt(jnp.maximum(dsum, 1.0))[:, 0:1]
    agg = (p_ref[0] + p_ref[1]) * dinv
    h = h_ref[...]
    logits = jnp.dot(h, wg_ref[...], preferred_element_type=jnp.float32)
    e_iota = lax.broadcasted_iota(jnp.int32, (BN, NEXP), 1)
    m1 = jnp.max(logits, axis=1, keepdims=True)
    i1 = jnp.min(jnp.where(logits == m1, e_iota, NEXP), axis=1, keepdims=True)
    masked = jnp.where(e_iota == i1, -jnp.inf, logits)
    m2 = jnp.max(masked, axis=1, keepdims=True)
    i2 = jnp.min(jnp.where(masked == m2, e_iota, NEXP), axis=1, keepdims=True)
    t = jnp.exp(m2 - m1)
    w1 = 1.0 / (1.0 + t)
    w2 = t / (1.0 + t)
    gates = (w1 * (e_iota == i1).astype(jnp.float32)
             + w2 * (e_iota == i2).astype(jnp.float32))
    gates_r = gates.astype(jnp.bfloat16).astype(jnp.float32)
    acc = jnp.zeros((BN, D), jnp.float32)
    for e in range(NEXP):
        eo = jnp.dot(agg, wx_ref[e], preferred_element_type=jnp.float32) + bx_ref[e]
        eo = eo.astype(jnp.bfloat16).astype(jnp.float32)
        acc = acc + gates_r[:, e:e + 1] * eo
    if relu:
        acc = jnp.maximum(acc, 0.0)
    h_out[...] = acc
    xs_out[...] = acc * dinv


def _make_layer(relu):
    return pl.pallas_call(
        functools.partial(_layer_body, relu=relu),
        grid=(NB,),
        in_specs=[
            pl.BlockSpec((BN, D), lambda i: (i, 0)),
            pl.BlockSpec((2, BN, D), lambda i: (0, i, 0)),
            pl.BlockSpec((2, BN, D), lambda i: (0, i, 0)),
            pl.BlockSpec((D, NEXP), lambda i: (0, 0)),
            pl.BlockSpec((NEXP, D, D), lambda i: (0, 0, 0)),
            pl.BlockSpec((NEXP, D), lambda i: (0, 0)),
        ],
        out_specs=[
            pl.BlockSpec((BN, D), lambda i: (i, 0)),
            pl.BlockSpec((BN, D), lambda i: (i, 0)),
        ],
        out_shape=[
            jax.ShapeDtypeStruct((N, D), jnp.float32),
            jax.ShapeDtypeStruct((N, D), jnp.float32),
        ],
    )


_layer_relu = _make_layer(True)
_layer_last = _make_layer(False)


def _pool_body(gid_ref, h_ref, w1_ref, b1_ref, w2_ref, b2_ref, w3_ref, b3_ref,
               w4_ref, b4_ref, wp_ref, bp_ref, out_logp, out_hidden, sums, cnts):
    i = pl.program_id(0)
    gcol = gid_ref[...]
    g_iota = lax.broadcasted_iota(jnp.int32, (BN, G), 1).astype(jnp.float32)
    oh = (gcol == g_iota).astype(jnp.float32)
    cdims = (((0,), (0,)), ((), ()))
    part = lax.dot_general(oh, h_ref[...], cdims,
                           preferred_element_type=jnp.float32,
                           precision=lax.Precision.HIGHEST)
    pcnt = lax.dot_general(oh, jnp.ones((BN, D), jnp.float32), cdims,
                           preferred_element_type=jnp.float32)

    @pl.when(i == 0)
    def _init():
        sums[...] = part
        cnts[...] = pcnt

    @pl.when(i > 0)
    def _accum():
        sums[...] += part
        cnts[...] += pcnt

    @pl.when(i == NB - 1)
    def _final():
        h_node = sums[...] / jnp.maximum(cnts[...], 1.0)
        hidden = jnp.maximum(
            jnp.dot(h_node, w1_ref[...], preferred_element_type=jnp.float32)
            + b1_ref[...], 0.0)
        h2 = jnp.maximum(
            jnp.dot(hidden, w2_ref[...], preferred_element_type=jnp.float32)
            + b2_ref[...], 0.0)
        h3 = jnp.maximum(
            jnp.dot(h2, w3_ref[...], preferred_element_type=jnp.float32)
            + b3_ref[...], 0.0)
        h4 = jnp.maximum(
            jnp.dot(h3, w4_ref[...], preferred_element_type=jnp.float32)
            + b4_ref[...], 0.0)
        y = jnp.dot(h4, wp_ref[...], preferred_element_type=jnp.float32) + bp_ref[...]
        m = jnp.max(y, axis=1, keepdims=True)
        lse = jnp.log(jnp.sum(jnp.exp(y - m), axis=1, keepdims=True)) + m
        out_logp[...] = y - lse
        out_hidden[...] = hidden


_pool_call = pl.pallas_call(
    _pool_body,
    grid=(NB,),
    in_specs=[
        pl.BlockSpec((BN, 1), lambda i: (i, 0)),
        pl.BlockSpec((BN, D), lambda i: (i, 0)),
        pl.BlockSpec((D, H), lambda i: (0, 0)),
        pl.BlockSpec((1, H), lambda i: (0, 0)),
        pl.BlockSpec((H, H), lambda i: (0, 0)),
        pl.BlockSpec((1, H), lambda i: (0, 0)),
        pl.BlockSpec((H, H), lambda i: (0, 0)),
        pl.BlockSpec((1, H), lambda i: (0, 0)),
        pl.BlockSpec((H, H), lambda i: (0, 0)),
        pl.BlockSpec((1, H), lambda i: (0, 0)),
        pl.BlockSpec((H, NT), lambda i: (0, 0)),
        pl.BlockSpec((1, NT), lambda i: (0, 0)),
    ],
    out_specs=[
        pl.BlockSpec((G, NT), lambda i: (0, 0)),
        pl.BlockSpec((G, H), lambda i: (0, 0)),
    ],
    out_shape=[
        jax.ShapeDtypeStruct((G, NT), jnp.float32),
        jax.ShapeDtypeStruct((G, H), jnp.float32),
    ],
    scratch_shapes=[
        pltpu.VMEM((G, D), jnp.float32),
        pltpu.VMEM((G, D), jnp.float32),
    ],
)


def kernel(x, edge_index, graph_ids, Wgate, Wexp, bexp,
           We1, be1, We2, be2, We3, be3, We4, be4, Wp, bp):
    src = edge_index[0]
    dst = edge_index[1]
    pad = E_PAD - E
    trash = TRASH + (jnp.arange(pad, dtype=jnp.int32) % 128)
    src_p = jnp.concatenate([src, jnp.zeros((pad,), jnp.int32)]).reshape(NW * 2, CHH, CB)
    dst_p = jnp.concatenate([dst, trash]).reshape(NW * 2, CHH, CB)
    ones_r = jnp.ones((CB, D), jnp.float32)
    zrows = jnp.zeros((CB, D), jnp.float32)

    deg_sc, agg_sc = _sc_kernels()
    deg_parts = deg_sc(dst_p, ones_r, zrows).reshape(2, NPAD, D)
    xs = _xs_call(deg_parts, x)

    h = x
    for l in range(L):
        parts = agg_sc(xs, src_p, dst_p, zrows).reshape(2, NPAD, D)
        layer = _layer_relu if l < L - 1 else _layer_last
        h, xs = layer(h, parts, deg_parts, Wgate[l], Wexp[l], bexp[l])

    gidf = graph_ids.astype(jnp.float32)[:, None]
    logp, hidden = _pool_call(gidf, h, We1, be1[None, :], We2, be2[None, :],
                              We3, be3[None, :], We4, be4[None, :], Wp, bp[None, :])
    return (logp, hidden)
